# Initial kernel scaffold; baseline (speedup 1.0000x reference)
import functools

import jax
import jax.numpy as jnp
from jax import lax
from jax.experimental import pallas as pl
from jax.experimental.pallas import tpu as pltpu
from jax.experimental.pallas import tpu_sc as plsc

N = 2048
E = 131072
D = 128
H = 128

NC = 2
NS = 16
RPT = N // NS
EW = E // (NC * NS)
K = 128
NB = EW // K


def _make_seg_sum(Dp):
    mesh = plsc.VectorSubcoreMesh(core_axis_name="c", subcore_axis_name="s")

    @functools.partial(
        pl.kernel,
        out_type=jax.ShapeDtypeStruct((NC, N, Dp), jnp.float32),
        mesh=mesh,
        scratch_types=[
            pltpu.VMEM((K,), jnp.int32),
            pltpu.VMEM((K,), jnp.int32),
            pltpu.VMEM((K, Dp), jnp.float32),
            pltpu.VMEM_SHARED((N, Dp), jnp.float32),
            pltpu.SemaphoreType.DMA,
        ],
    )
    def seg_sum(table, ei, zeros, out, src_v, dst_v, rows_v, acc, sem):
        cid = lax.axis_index("c")
        sid = lax.axis_index("s")
        wid = cid * NS + sid
        pltpu.sync_copy(zeros.at[pl.ds(sid * RPT, RPT)],
                        acc.at[pl.ds(sid * RPT, RPT)])
        plsc.subcore_barrier()

        def body(i, carry):
            base = pl.multiple_of(wid * EW + i * K, K)
            pltpu.sync_copy(ei.at[0, pl.ds(base, K)], src_v)
            pltpu.sync_copy(ei.at[1, pl.ds(base, K)], dst_v)
            pltpu.async_copy(table.at[src_v], rows_v, sem).wait()
            pltpu.sync_copy(rows_v, acc.at[dst_v], add=True)
            return carry

        lax.fori_loop(0, NB, body, 0)
        plsc.subcore_barrier()
        pltpu.sync_copy(acc.at[pl.ds(sid * RPT, RPT)],
                        out.at[cid, pl.ds(sid * RPT, RPT)])

    return seg_sum


_seg144 = _make_seg_sum(D + 16)
_seg256 = _make_seg_sum(2 * H)
_seg128 = _make_seg_sum(H)


def _relu(v):
    return jnp.maximum(v, 0.0)


def _dot(a, b):
    return jnp.dot(a, b, preferred_element_type=jnp.float32)


def _t1_body(part, x, wl, wr, b, g1w, u1_o, dinv_o):
    s = part[0] + part[1]
    cnt = s[:, D:D + 1]
    agg = s[:, :D] / jnp.maximum(cnt, 1.0)
    h1 = _relu(_dot(agg, wl[:]) + _dot(x[:], wr[:]) + b[:])
    dinv = lax.rsqrt(cnt + 1.0)
    dinv_o[:] = dinv
    u1_o[:] = dinv * _dot(h1, g1w[:])


def _t2_body(part, u1, dinv, g1b, g2w, u2_o):
    s = part[0] + part[1]
    h2 = _relu(dinv[:] * (s + u1[:]) + g1b[:])
    u2_o[:] = dinv[:] * _dot(h2, g2w[:])


CB = 1024


def _t3_body(part, u2, dinv, g2b, f1w, f1b, f2w, f2b, ow, ob, out_o, h5):
    @pl.when(pl.program_id(0) == 0)
    def _():
        s = part[0] + part[1]
        h3 = _relu(dinv[:] * (s + u2[:]) + g2b[:])
        h4 = _relu(_dot(h3, f1w[:]) + f1b[:])
        h5[:] = _relu(_dot(h4, f2w[:]) + f2b[:])

    out_o[:] = _dot(h5[:], ow[:]) + ob[:]


def _full2(i):
    del i
    return 0, 0


def _full3(i):
    del i
    return 0, 0, 0


def _t1_call(part, x, wl, wr, b, g1w):
    return pl.pallas_call(
        _t1_body,
        out_shape=(jax.ShapeDtypeStruct((N, 2 * H), jnp.float32),
                   jax.ShapeDtypeStruct((N, 1), jnp.float32)),
    )(part, x, wl, wr, b, g1w)


def _t2_call(part, u1, dinv, g1b, g2w):
    return pl.pallas_call(
        _t2_body,
        out_shape=jax.ShapeDtypeStruct((N, H), jnp.float32),
    )(part, u1, dinv, g1b, g2w)


def _t3_call(part, u2, dinv, g2b, f1w, f1b, f2w, f2b, ow, ob):
    nblk = 4 * N // CB
    return pl.pallas_call(
        _t3_body,
        grid=(nblk,),
        in_specs=[
            pl.BlockSpec((NC, N, H), _full3),
            pl.BlockSpec((N, H), _full2),
            pl.BlockSpec((N, 1), _full2),
            pl.BlockSpec((1, H), _full2),
            pl.BlockSpec((H, H), _full2),
            pl.BlockSpec((1, H), _full2),
            pl.BlockSpec((H, H), _full2),
            pl.BlockSpec((1, H), _full2),
            pl.BlockSpec((H, CB), lambda i: (0, i)),
            pl.BlockSpec((1, CB), lambda i: (0, i)),
        ],
        out_specs=pl.BlockSpec((N, CB), lambda i: (0, i)),
        out_shape=jax.ShapeDtypeStruct((N, 4 * N), jnp.float32),
        scratch_shapes=[pltpu.VMEM((N, H), jnp.float32)],
    )(part, u2, dinv, g2b, f1w, f1b, f2w, f2b, ow, ob)


def kernel(x, edge_index, sage_wl, sage_wr, sage_b, gcn1_w, gcn1_b,
           gcn2_w, gcn2_b, fc1_w, fc1_b, fc2_w, fc2_b, out_w, out_b):
    xext = jnp.concatenate([x, jnp.ones((N, 16), jnp.float32)], axis=1)
    part1 = _seg144(xext, edge_index, jnp.zeros((N, D + 16), jnp.float32))
    u1, dinv = _t1_call(part1, x, sage_wl, sage_wr,
                        sage_b.reshape(1, 2 * H), gcn1_w)
    part2 = _seg256(u1, edge_index, jnp.zeros((N, 2 * H), jnp.float32))
    u2 = _t2_call(part2, u1, dinv, gcn1_b.reshape(1, 2 * H), gcn2_w)
    part3 = _seg128(u2, edge_index, jnp.zeros((N, H), jnp.float32))
    out = _t3_call(part3, u2, dinv, gcn2_b.reshape(1, H),
                   fc1_w, fc1_b.reshape(1, H), fc2_w, fc2_b.reshape(1, H),
                   out_w, out_b.reshape(1, 4 * N))
    return out.reshape(N, 4, N)


# speedup vs baseline: 1.7750x; 1.7750x over previous
import functools

import jax
import jax.numpy as jnp
from jax import lax
from jax.experimental import pallas as pl
from jax.experimental.pallas import tpu as pltpu
from jax.experimental.pallas import tpu_sc as plsc

N = 2048
E = 131072
D = 128
H = 128

NC = 2
NS = 16
RPT = N // NS
EW = E // (NC * NS)
K = 128
NB = EW // K


def _make_seg_sum(Dp):
    mesh = plsc.VectorSubcoreMesh(core_axis_name="c", subcore_axis_name="s",
                                  num_cores=NC, num_subcores=NS)

    @functools.partial(
        pl.kernel,
        out_type=jax.ShapeDtypeStruct((NC, N, Dp), jnp.float32),
        mesh=mesh,
        scratch_types=[
            pltpu.VMEM((K,), jnp.int32),
            pltpu.VMEM((K,), jnp.int32),
            pltpu.VMEM((K, Dp), jnp.float32),
            pltpu.VMEM_SHARED((N, Dp), jnp.float32),
            pltpu.SemaphoreType.DMA,
        ],
        compiler_params=pltpu.CompilerParams(needs_layout_passes=False),
    )
    def seg_sum(table, ei, zeros, out, src_v, dst_v, rows_v, acc, sem):
        cid = lax.axis_index("c")
        sid = lax.axis_index("s")
        wid = cid * NS + sid
        pltpu.sync_copy(zeros.at[pl.ds(sid * RPT, RPT)],
                        acc.at[pl.ds(sid * RPT, RPT)])
        plsc.subcore_barrier()

        def body(i, carry):
            base = pl.multiple_of(wid * EW + i * K, K)
            pltpu.sync_copy(ei.at[0, pl.ds(base, K)], src_v)
            pltpu.sync_copy(ei.at[1, pl.ds(base, K)], dst_v)
            pltpu.async_copy(table.at[src_v], rows_v, sem).wait()
            pltpu.sync_copy(rows_v, acc.at[dst_v], add=True)
            return carry

        lax.fori_loop(0, NB, body, 0)
        plsc.subcore_barrier()
        pltpu.sync_copy(acc.at[pl.ds(sid * RPT, RPT)],
                        out.at[cid, pl.ds(sid * RPT, RPT)])

    return seg_sum


def _make_sage_seg():
    mesh = plsc.VectorSubcoreMesh(core_axis_name="c", subcore_axis_name="s",
                                  num_cores=NC, num_subcores=NS)

    @functools.partial(
        pl.kernel,
        out_type=(jax.ShapeDtypeStruct((NC, N, D), jnp.float32),
                  jax.ShapeDtypeStruct((NC * NS, N), jnp.float32)),
        mesh=mesh,
        scratch_types=[
            pltpu.VMEM((K,), jnp.int32),
            pltpu.VMEM((K,), jnp.int32),
            pltpu.VMEM((K, D), jnp.float32),
            pltpu.VMEM((N,), jnp.float32),
            pltpu.VMEM_SHARED((N, D), jnp.float32),
            pltpu.SemaphoreType.DMA,
        ],
        compiler_params=pltpu.CompilerParams(needs_layout_passes=False),
    )
    def sage_seg(table, ei, zeros, out, out_cnt,
                 src_v, dst_v, rows_v, cnt_v, acc, sem):
        cid = lax.axis_index("c")
        sid = lax.axis_index("s")
        wid = cid * NS + sid
        pltpu.sync_copy(zeros.at[pl.ds(sid * RPT, RPT)],
                        acc.at[pl.ds(sid * RPT, RPT)])
        z16 = jnp.zeros((16,), jnp.float32)

        def zbody(j, carry):
            cnt_v[pl.ds(j * 16, 16)] = z16
            return carry

        lax.fori_loop(0, N // 16, zbody, 0)
        plsc.subcore_barrier()
        ones16 = jnp.ones((16,), jnp.float32)

        def body(i, carry):
            base = pl.multiple_of(wid * EW + i * K, K)
            pltpu.sync_copy(ei.at[0, pl.ds(base, K)], src_v)
            pltpu.sync_copy(ei.at[1, pl.ds(base, K)], dst_v)
            pltpu.async_copy(table.at[src_v], rows_v, sem).wait()
            pltpu.sync_copy(rows_v, acc.at[dst_v], add=True)
            for j in range(K // 16):
                d16 = dst_v[pl.ds(j * 16, 16)]
                plsc.addupdate_scatter(cnt_v, [d16], ones16)
            return carry

        lax.fori_loop(0, NB, body, 0)
        plsc.subcore_barrier()
        pltpu.sync_copy(acc.at[pl.ds(sid * RPT, RPT)],
                        out.at[cid, pl.ds(sid * RPT, RPT)])
        pltpu.sync_copy(cnt_v, out_cnt.at[wid])

    return sage_seg


_seg_cache = {}


def _seg(Dp):
    if Dp not in _seg_cache:
        _seg_cache[Dp] = _make_seg_sum(Dp)
    return _seg_cache[Dp]


def _sage_seg():
    if "sage" not in _seg_cache:
        _seg_cache["sage"] = _make_sage_seg()
    return _seg_cache["sage"]


def _relu(v):
    return jnp.maximum(v, 0.0)


def _dot(a, b):
    return jnp.dot(a, b, preferred_element_type=jnp.float32)


def _t1_body(part, cnts, x, wl, wr, b, g1w, u1_o, dinv_o):
    s = part[0] + part[1]
    cnt = lax.dot_general(cnts[:], jnp.ones((NC * NS, 1), jnp.float32),
                          (((0,), (0,)), ((), ())),
                          preferred_element_type=jnp.float32)
    agg = s / jnp.maximum(cnt, 1.0)
    h1 = _relu(_dot(agg, wl[:]) + _dot(x[:], wr[:]) + b[:])
    dinv = lax.rsqrt(cnt + 1.0)
    dinv_o[:] = dinv
    u1_o[:] = dinv * _dot(h1, g1w[:])


def _t2_body(part, u1, dinv, g1b, g2w, u2_o):
    s = part[0] + part[1]
    h2 = _relu(dinv[:] * (s + u1[:]) + g1b[:])
    u2_o[:] = dinv[:] * _dot(h2, g2w[:])


CB = 1024


def _t3_body(part, u2, dinv, g2b, f1w, f1b, f2w, f2b, ow, ob, out_o, h5):
    @pl.when(pl.program_id(0) == 0)
    def _():
        s = part[0] + part[1]
        h3 = _relu(dinv[:] * (s + u2[:]) + g2b[:])
        h4 = _relu(_dot(h3, f1w[:]) + f1b[:])
        h5[:] = _relu(_dot(h4, f2w[:]) + f2b[:])

    out_o[:] = _dot(h5[:], ow[:]) + ob[:]


def _full2(i):
    del i
    return 0, 0


def _full3(i):
    del i
    return 0, 0, 0


def _t1_call(part, cnts, x, wl, wr, b, g1w):
    return pl.pallas_call(
        _t1_body,
        out_shape=(jax.ShapeDtypeStruct((N, 2 * H), jnp.float32),
                   jax.ShapeDtypeStruct((N, 1), jnp.float32)),
    )(part, cnts, x, wl, wr, b, g1w)


def _t2_call(part, u1, dinv, g1b, g2w):
    return pl.pallas_call(
        _t2_body,
        out_shape=jax.ShapeDtypeStruct((N, H), jnp.float32),
    )(part, u1, dinv, g1b, g2w)


def _t3_call(part, u2, dinv, g2b, f1w, f1b, f2w, f2b, ow, ob):
    nblk = 4 * N // CB
    return pl.pallas_call(
        _t3_body,
        grid=(nblk,),
        in_specs=[
            pl.BlockSpec((NC, N, H), _full3),
            pl.BlockSpec((N, H), _full2),
            pl.BlockSpec((N, 1), _full2),
            pl.BlockSpec((1, H), _full2),
            pl.BlockSpec((H, H), _full2),
            pl.BlockSpec((1, H), _full2),
            pl.BlockSpec((H, H), _full2),
            pl.BlockSpec((1, H), _full2),
            pl.BlockSpec((H, CB), lambda i: (0, i)),
            pl.BlockSpec((1, CB), lambda i: (0, i)),
        ],
        out_specs=pl.BlockSpec((N, CB), lambda i: (0, i)),
        out_shape=jax.ShapeDtypeStruct((N, 4 * N), jnp.float32),
        scratch_shapes=[pltpu.VMEM((N, H), jnp.float32)],
    )(part, u2, dinv, g2b, f1w, f1b, f2w, f2b, ow, ob)


def _tmp_seg(table, ei, zeros):
    del zeros
    src, dst = ei[0], ei[1]
    parts = []
    half = E // NC
    for c in range(NC):
        s = src[c * half:(c + 1) * half]
        d = dst[c * half:(c + 1) * half]
        parts.append(jax.ops.segment_sum(table[s], d, num_segments=N))
    return jnp.stack(parts)


def _tmp_sage(table, ei, zeros):
    parts = _tmp_seg(table, ei, zeros)
    dst = ei[1]
    cnts = []
    for w in range(NC * NS):
        d = dst[w * EW:(w + 1) * EW]
        cnts.append(jax.ops.segment_sum(jnp.ones_like(d, jnp.float32), d,
                                        num_segments=N))
    return parts, jnp.stack(cnts)


def kernel(x, edge_index, sage_wl, sage_wr, sage_b, gcn1_w, gcn1_b,
           gcn2_w, gcn2_b, fc1_w, fc1_b, fc2_w, fc2_b, out_w, out_b):
    part1, cnts = _tmp_sage(x, edge_index, jnp.zeros((N, D), jnp.float32))
    u1, dinv = _t1_call(part1, cnts, x, sage_wl, sage_wr,
                        sage_b.reshape(1, 2 * H), gcn1_w)
    part2 = _tmp_seg(u1, edge_index, jnp.zeros((N, 2 * H), jnp.float32))
    u2 = _t2_call(part2, u1, dinv, gcn1_b.reshape(1, 2 * H), gcn2_w)
    part3 = _tmp_seg(u2, edge_index, jnp.zeros((N, H), jnp.float32))
    out = _t3_call(part3, u2, dinv, gcn2_b.reshape(1, H),
                   fc1_w, fc1_b.reshape(1, H), fc2_w, fc2_b.reshape(1, H),
                   out_w, out_b.reshape(1, 4 * N))
    return out.reshape(N, 4, N)


# speedup vs baseline: 3.0546x; 1.7209x over previous
import functools

import jax
import jax.numpy as jnp
from jax import lax
from jax.experimental import pallas as pl
from jax.experimental.pallas import tpu as pltpu
from jax.experimental.pallas import tpu_sc as plsc

N = 2048
E = 131072
D = 128
H = 128

NC = 2
NS = 16
EH = E // NC
EBATCH = 16384
NBATCH = EH // EBATCH
NG = EBATCH // 16
ECNT = EH // NS


def _make_seg_sum(Dp, with_counts):
    CW = Dp // NS
    FL = N * CW
    mesh = plsc.VectorSubcoreMesh(core_axis_name="c", subcore_axis_name="s",
                                  num_cores=NC, num_subcores=NS)
    out_types = (jax.ShapeDtypeStruct((NC, NS, FL), jnp.float32),)
    scratch = [
        pltpu.VMEM((EBATCH,), jnp.int32),
        pltpu.VMEM((EBATCH,), jnp.int32),
        pltpu.VMEM((FL,), jnp.float32),
        pltpu.VMEM((FL,), jnp.float32),
    ]
    if with_counts:
        out_types += (jax.ShapeDtypeStruct((NC * NS, N), jnp.float32),)
        scratch.append(pltpu.VMEM((N,), jnp.float32))

    @functools.partial(
        pl.kernel,
        out_type=out_types if with_counts else out_types[0],
        mesh=mesh,
        scratch_types=scratch,
        compiler_params=pltpu.CompilerParams(needs_layout_passes=False),
    )
    def seg_sum(tab_t, ei, zeros, out, *rest):
        if with_counts:
            out_cnt, src_v, dst_v, tab_v, acc_v, cnt_v = rest
        else:
            src_v, dst_v, tab_v, acc_v = rest
        cid = lax.axis_index("c")
        sid = lax.axis_index("s")
        pltpu.sync_copy(tab_t.at[sid], tab_v)
        pltpu.sync_copy(zeros, acc_v)
        for b in range(NBATCH):
            base = pl.multiple_of(cid * EH + b * EBATCH, EBATCH)
            pltpu.sync_copy(ei.at[0, pl.ds(base, EBATCH)], src_v)
            pltpu.sync_copy(ei.at[1, pl.ds(base, EBATCH)], dst_v)

            def gbody(g, carry):
                s16 = src_v[pl.ds(g * 16, 16)]
                d16 = dst_v[pl.ds(g * 16, 16)]
                sb = s16 * CW
                db = d16 * CW
                for c in range(CW):
                    vals = plsc.load_gather(tab_v, [sb + c])
                    plsc.addupdate_scatter(acc_v, [db + c], vals)
                return carry

            lax.fori_loop(0, NG, gbody, 0)
        if with_counts:
            pltpu.sync_copy(zeros.at[pl.ds(0, N)], cnt_v)
            cb = pl.multiple_of(cid * EH + sid * ECNT, ECNT)
            pltpu.sync_copy(ei.at[1, pl.ds(cb, ECNT)],
                            dst_v.at[pl.ds(0, ECNT)])
            ones16 = jnp.ones((16,), jnp.float32)

            def cbody(g, carry):
                d16 = dst_v[pl.ds(g * 16, 16)]
                plsc.addupdate_scatter(cnt_v, [d16], ones16)
                return carry

            lax.fori_loop(0, ECNT // 16, cbody, 0)
            wid = cid * NS + sid
            pltpu.sync_copy(cnt_v, out_cnt.at[wid])
        pltpu.sync_copy(acc_v, out.at[cid, sid])

    return seg_sum


_seg_cache = {}


def _seg(Dp, with_counts=False):
    key = (Dp, with_counts)
    if key not in _seg_cache:
        _seg_cache[key] = _make_seg_sum(Dp, with_counts)
    return _seg_cache[key]


def _to_tiles(table, Dp):
    cw = Dp // NS
    return table.reshape(N, NS, cw).transpose(1, 0, 2).reshape(NS, N * cw)


def _from_tiles(out, Dp):
    cw = Dp // NS
    r = out.reshape(NC, NS, N, cw).transpose(0, 2, 1, 3).reshape(NC, N, Dp)
    return r[0], r[1]


def _relu(v):
    return jnp.maximum(v, 0.0)


def _dot(a, b):
    return jnp.dot(a, b, preferred_element_type=jnp.float32)


def _t1_body(r0, r1, cnts, x, wl, wr, b, g1w, u1_o, dinv_o):
    s = r0[:] + r1[:]
    cnt = lax.dot_general(cnts[:], jnp.ones((NC * NS, 1), jnp.float32),
                          (((0,), (0,)), ((), ())),
                          preferred_element_type=jnp.float32)
    agg = s / jnp.maximum(cnt, 1.0)
    h1 = _relu(_dot(agg, wl[:]) + _dot(x[:], wr[:]) + b[:])
    dinv = lax.rsqrt(cnt + 1.0)
    dinv_o[:] = dinv
    u1_o[:] = dinv * _dot(h1, g1w[:])


def _t2_body(r0, r1, u1, dinv, g1b, g2w, u2_o):
    s = r0[:] + r1[:]
    h2 = _relu(dinv[:] * (s + u1[:]) + g1b[:])
    u2_o[:] = dinv[:] * _dot(h2, g2w[:])


CB = 1024


def _t3_body(r0, r1, u2, dinv, g2b, f1w, f1b, f2w, f2b, ow, ob, out_o, h5):
    @pl.when(pl.program_id(0) == 0)
    def _():
        s = r0[:] + r1[:]
        h3 = _relu(dinv[:] * (s + u2[:]) + g2b[:])
        h4 = _relu(_dot(h3, f1w[:]) + f1b[:])
        h5[:] = _relu(_dot(h4, f2w[:]) + f2b[:])

    out_o[:] = _dot(h5[:], ow[:]) + ob[:]


def _full2(i):
    del i
    return 0, 0


def _t1_call(r0, r1, cnts, x, wl, wr, b, g1w):
    return pl.pallas_call(
        _t1_body,
        out_shape=(jax.ShapeDtypeStruct((N, 2 * H), jnp.float32),
                   jax.ShapeDtypeStruct((N, 1), jnp.float32)),
    )(r0, r1, cnts, x, wl, wr, b, g1w)


def _t2_call(r0, r1, u1, dinv, g1b, g2w):
    return pl.pallas_call(
        _t2_body,
        out_shape=jax.ShapeDtypeStruct((N, H), jnp.float32),
    )(r0, r1, u1, dinv, g1b, g2w)


def _t3_call(r0, r1, u2, dinv, g2b, f1w, f1b, f2w, f2b, ow, ob):
    nblk = 4 * N // CB
    return pl.pallas_call(
        _t3_body,
        grid=(nblk,),
        in_specs=[
            pl.BlockSpec((N, H), _full2),
            pl.BlockSpec((N, H), _full2),
            pl.BlockSpec((N, H), _full2),
            pl.BlockSpec((N, 1), _full2),
            pl.BlockSpec((1, H), _full2),
            pl.BlockSpec((H, H), _full2),
            pl.BlockSpec((1, H), _full2),
            pl.BlockSpec((H, H), _full2),
            pl.BlockSpec((1, H), _full2),
            pl.BlockSpec((H, CB), lambda i: (0, i)),
            pl.BlockSpec((1, CB), lambda i: (0, i)),
        ],
        out_specs=pl.BlockSpec((N, CB), lambda i: (0, i)),
        out_shape=jax.ShapeDtypeStruct((N, 4 * N), jnp.float32),
        scratch_shapes=[pltpu.VMEM((N, H), jnp.float32)],
    )(r0, r1, u2, dinv, g2b, f1w, f1b, f2w, f2b, ow, ob)


def kernel(x, edge_index, sage_wl, sage_wr, sage_b, gcn1_w, gcn1_b,
           gcn2_w, gcn2_b, fc1_w, fc1_b, fc2_w, fc2_b, out_w, out_b):
    z128 = jnp.zeros((N * (D // NS),), jnp.float32)
    z256 = jnp.zeros((N * (2 * H // NS),), jnp.float32)
    p1, cnts = _seg(D, True)(_to_tiles(x, D), edge_index, z128)
    r0, r1 = _from_tiles(p1, D)
    u1, dinv = _t1_call(r0, r1, cnts, x, sage_wl, sage_wr,
                        sage_b.reshape(1, 2 * H), gcn1_w)
    p2 = _seg(2 * H)(_to_tiles(u1, 2 * H), edge_index, z256)
    r0, r1 = _from_tiles(p2, 2 * H)
    u2 = _t2_call(r0, r1, u1, dinv, gcn1_b.reshape(1, 2 * H), gcn2_w)
    p3 = _seg(H)(_to_tiles(u2, H), edge_index, z128)
    r0, r1 = _from_tiles(p3, H)
    out = _t3_call(r0, r1, u2, dinv, gcn2_b.reshape(1, H),
                   fc1_w, fc1_b.reshape(1, H), fc2_w, fc2_b.reshape(1, H),
                   out_w, out_b.reshape(1, 4 * N))
    return out.reshape(N, 4, N)


# speedup vs baseline: 16.0934x; 5.2686x over previous
import functools

import jax
import jax.numpy as jnp
from jax import lax
from jax.experimental import pallas as pl
from jax.experimental.pallas import tpu as pltpu
from jax.experimental.pallas import tpu_sc as plsc

N = 2048
E = 131072
D = 128
H = 128

NC = 2
NS = 16
CW = 128
RH = 512
NCC = N // CW
NRC = N // RH
EBATCH = 16384
NBATCH = E // EBATCH
NG = EBATCH // 16


def _make_adj():
    mesh = plsc.VectorSubcoreMesh(core_axis_name="c", subcore_axis_name="s",
                                  num_cores=NC, num_subcores=NS)

    @functools.partial(
        pl.kernel,
        out_type=jax.ShapeDtypeStruct((N, N), jnp.float32),
        mesh=mesh,
        scratch_types=[
            pltpu.VMEM((EBATCH,), jnp.int32),
            pltpu.VMEM((EBATCH,), jnp.int32),
            pltpu.VMEM((RH, CW), jnp.float32),
        ],
        compiler_params=pltpu.CompilerParams(needs_layout_passes=False),
    )
    def adj(ei, zeros, out, src_v, dst_v, acc_v):
        cid = lax.axis_index("c")
        sid = lax.axis_index("s")
        wid = cid * NS + sid
        ones16 = jnp.ones((16,), jnp.float32)
        for p in range(2):
            combo = wid + 32 * p
            clo = pl.multiple_of((combo % NCC) * CW, CW)
            rlo = pl.multiple_of((combo // NCC) * RH, RH)
            pltpu.sync_copy(zeros, acc_v)
            for b in range(NBATCH):
                base = pl.multiple_of(b * EBATCH, EBATCH)
                pltpu.sync_copy(ei.at[0, pl.ds(base, EBATCH)], src_v)
                pltpu.sync_copy(ei.at[1, pl.ds(base, EBATCH)], dst_v)

                def gbody(g, carry):
                    s16 = src_v[pl.ds(g * 16, 16)]
                    d16 = dst_v[pl.ds(g * 16, 16)]
                    sc = s16 - clo
                    dr = d16 - rlo
                    m = ((sc >= 0) & (sc < CW)) & ((dr >= 0) & (dr < RH))
                    plsc.addupdate_scatter(acc_v, [dr, sc], ones16, mask=m)
                    return carry

                lax.fori_loop(0, NG, gbody, 0)
            pltpu.sync_copy(acc_v, out.at[pl.ds(rlo, RH), pl.ds(clo, CW)])

    return adj


_adj_cache = {}


def _adj():
    if "adj" not in _adj_cache:
        _adj_cache["adj"] = _make_adj()
    return _adj_cache["adj"]


def _relu(v):
    return jnp.maximum(v, 0.0)


def _dot(a, b):
    return jnp.dot(a, b, preferred_element_type=jnp.float32)


_VMEM_PARAMS = pltpu.CompilerParams(vmem_limit_bytes=100 * 1024 * 1024)


def _t1_body(adj, x, wl, wr, b, g1w, u1_o, dinv_o):
    s = _dot(adj[:], x[:])
    cnt = jnp.sum(adj[:], axis=1, keepdims=True)
    agg = s / jnp.maximum(cnt, 1.0)
    h1 = _relu(_dot(agg, wl[:]) + _dot(x[:], wr[:]) + b[:])
    dinv = lax.rsqrt(cnt + 1.0)
    dinv_o[:] = dinv
    u1_o[:] = dinv * _dot(h1, g1w[:])


def _t2_body(adj, u1, dinv, g1b, g2w, u2_o):
    s = _dot(adj[:], u1[:])
    h2 = _relu(dinv[:] * (s + u1[:]) + g1b[:])
    u2_o[:] = dinv[:] * _dot(h2, g2w[:])


CB = 1024


def _t3_body(adj, u2, dinv, g2b, f1w, f1b, f2w, f2b, ow, ob, out_o, h5):
    @pl.when(pl.program_id(0) == 0)
    def _():
        s = _dot(adj[:], u2[:])
        h3 = _relu(dinv[:] * (s + u2[:]) + g2b[:])
        h4 = _relu(_dot(h3, f1w[:]) + f1b[:])
        h5[:] = _relu(_dot(h4, f2w[:]) + f2b[:])

    out_o[:] = _dot(h5[:], ow[:]) + ob[:]


def _full2(i):
    del i
    return 0, 0


def _t1_call(adj, x, wl, wr, b, g1w):
    return pl.pallas_call(
        _t1_body,
        out_shape=(jax.ShapeDtypeStruct((N, 2 * H), jnp.float32),
                   jax.ShapeDtypeStruct((N, 1), jnp.float32)),
        compiler_params=_VMEM_PARAMS,
    )(adj, x, wl, wr, b, g1w)


def _t2_call(adj, u1, dinv, g1b, g2w):
    return pl.pallas_call(
        _t2_body,
        out_shape=jax.ShapeDtypeStruct((N, H), jnp.float32),
        compiler_params=_VMEM_PARAMS,
    )(adj, u1, dinv, g1b, g2w)


def _t3_call(adj, u2, dinv, g2b, f1w, f1b, f2w, f2b, ow, ob):
    nblk = 4 * N // CB
    return pl.pallas_call(
        _t3_body,
        grid=(nblk,),
        in_specs=[
            pl.BlockSpec((N, N), _full2),
            pl.BlockSpec((N, H), _full2),
            pl.BlockSpec((N, 1), _full2),
            pl.BlockSpec((1, H), _full2),
            pl.BlockSpec((H, H), _full2),
            pl.BlockSpec((1, H), _full2),
            pl.BlockSpec((H, H), _full2),
            pl.BlockSpec((1, H), _full2),
            pl.BlockSpec((H, CB), lambda i: (0, i)),
            pl.BlockSpec((1, CB), lambda i: (0, i)),
        ],
        out_specs=pl.BlockSpec((N, CB), lambda i: (0, i)),
        out_shape=jax.ShapeDtypeStruct((N, 4 * N), jnp.float32),
        scratch_shapes=[pltpu.VMEM((N, H), jnp.float32)],
        compiler_params=_VMEM_PARAMS,
    )(adj, u2, dinv, g2b, f1w, f1b, f2w, f2b, ow, ob)


def kernel(x, edge_index, sage_wl, sage_wr, sage_b, gcn1_w, gcn1_b,
           gcn2_w, gcn2_b, fc1_w, fc1_b, fc2_w, fc2_b, out_w, out_b):
    adj = _adj()(edge_index, jnp.zeros((RH, CW), jnp.float32))
    u1, dinv = _t1_call(adj, x, sage_wl, sage_wr,
                        sage_b.reshape(1, 2 * H), gcn1_w)
    u2 = _t2_call(adj, u1, dinv, gcn1_b.reshape(1, 2 * H), gcn2_w)
    out = _t3_call(adj, u2, dinv, gcn2_b.reshape(1, H),
                   fc1_w, fc1_b.reshape(1, H), fc2_w, fc2_b.reshape(1, H),
                   out_w, out_b.reshape(1, 4 * N))
    return out.reshape(N, 4, N)


# speedup vs baseline: 22.7289x; 1.4123x over previous
import functools

import jax
import jax.numpy as jnp
from jax import lax
from jax.experimental import pallas as pl
from jax.experimental.pallas import tpu as pltpu
from jax.experimental.pallas import tpu_sc as plsc

N = 2048
E = 131072
D = 128
H = 128

NC = 2
NS = 16
CW = 128
RH = 512
NCC = N // CW
NRC = N // RH
EBATCH = 16384
NBATCH = E // EBATCH
NG = EBATCH // 16


def _make_adj():
    mesh = plsc.VectorSubcoreMesh(core_axis_name="c", subcore_axis_name="s",
                                  num_cores=NC, num_subcores=NS)

    @functools.partial(
        pl.kernel,
        out_type=jax.ShapeDtypeStruct((N, N), jnp.float32),
        mesh=mesh,
        scratch_types=[
            pltpu.VMEM((EBATCH,), jnp.int32),
            pltpu.VMEM((EBATCH,), jnp.int32),
            pltpu.VMEM((RH, CW), jnp.float32),
        ],
        compiler_params=pltpu.CompilerParams(needs_layout_passes=False),
    )
    def adj(ei, zeros, out, src_v, dst_v, acc_v):
        cid = lax.axis_index("c")
        sid = lax.axis_index("s")
        wid = cid * NS + sid
        ones16 = jnp.ones((16,), jnp.float32)
        for p in range(2):
            combo = wid + 32 * p
            clo = pl.multiple_of((combo % NCC) * CW, CW)
            rlo = pl.multiple_of((combo // NCC) * RH, RH)
            pltpu.sync_copy(zeros, acc_v)
            for b in range(NBATCH):
                base = pl.multiple_of(b * EBATCH, EBATCH)
                pltpu.sync_copy(ei.at[0, pl.ds(base, EBATCH)], src_v)
                pltpu.sync_copy(ei.at[1, pl.ds(base, EBATCH)], dst_v)

                @plsc.parallel_loop(0, NG, unroll=8)
                def _(g):
                    s16 = src_v[pl.ds(g * 16, 16)]
                    d16 = dst_v[pl.ds(g * 16, 16)]
                    sc = s16 - clo
                    dr = d16 - rlo
                    m = ((sc >= 0) & (sc < CW)) & ((dr >= 0) & (dr < RH))
                    plsc.addupdate_scatter(acc_v, [dr, sc], ones16, mask=m)
            pltpu.sync_copy(acc_v, out.at[pl.ds(rlo, RH), pl.ds(clo, CW)])

    return adj


_adj_cache = {}


def _adj():
    if "adj" not in _adj_cache:
        _adj_cache["adj"] = _make_adj()
    return _adj_cache["adj"]


def _relu(v):
    return jnp.maximum(v, 0.0)


def _dot(a, b):
    return jnp.dot(a, b, preferred_element_type=jnp.float32)


_VMEM_PARAMS = pltpu.CompilerParams(vmem_limit_bytes=100 * 1024 * 1024)


def _t1_body(adj, x, wl, wr, b, g1w, u1_o, dinv_o):
    s = _dot(adj[:], x[:])
    cnt = jnp.sum(adj[:], axis=1, keepdims=True)
    agg = s / jnp.maximum(cnt, 1.0)
    h1 = _relu(_dot(agg, wl[:]) + _dot(x[:], wr[:]) + b[:])
    dinv = lax.rsqrt(cnt + 1.0)
    dinv_o[:] = dinv
    u1_o[:] = dinv * _dot(h1, g1w[:])


def _t2_body(adj, u1, dinv, g1b, g2w, u2_o):
    s = _dot(adj[:], u1[:])
    h2 = _relu(dinv[:] * (s + u1[:]) + g1b[:])
    u2_o[:] = dinv[:] * _dot(h2, g2w[:])


CB = 1024


def _t3_body(adj, u2, dinv, g2b, f1w, f1b, f2w, f2b, ow, ob, out_o, h5):
    @pl.when(pl.program_id(0) == 0)
    def _():
        s = _dot(adj[:], u2[:])
        h3 = _relu(dinv[:] * (s + u2[:]) + g2b[:])
        h4 = _relu(_dot(h3, f1w[:]) + f1b[:])
        h5[:] = _relu(_dot(h4, f2w[:]) + f2b[:])

    out_o[:] = _dot(h5[:], ow[:]) + ob[:]


def _full2(i):
    del i
    return 0, 0


def _t1_call(adj, x, wl, wr, b, g1w):
    return pl.pallas_call(
        _t1_body,
        out_shape=(jax.ShapeDtypeStruct((N, 2 * H), jnp.float32),
                   jax.ShapeDtypeStruct((N, 1), jnp.float32)),
        compiler_params=_VMEM_PARAMS,
    )(adj, x, wl, wr, b, g1w)


def _t2_call(adj, u1, dinv, g1b, g2w):
    return pl.pallas_call(
        _t2_body,
        out_shape=jax.ShapeDtypeStruct((N, H), jnp.float32),
        compiler_params=_VMEM_PARAMS,
    )(adj, u1, dinv, g1b, g2w)


def _t3_call(adj, u2, dinv, g2b, f1w, f1b, f2w, f2b, ow, ob):
    nblk = 4 * N // CB
    return pl.pallas_call(
        _t3_body,
        grid=(nblk,),
        in_specs=[
            pl.BlockSpec((N, N), _full2),
            pl.BlockSpec((N, H), _full2),
            pl.BlockSpec((N, 1), _full2),
            pl.BlockSpec((1, H), _full2),
            pl.BlockSpec((H, H), _full2),
            pl.BlockSpec((1, H), _full2),
            pl.BlockSpec((H, H), _full2),
            pl.BlockSpec((1, H), _full2),
            pl.BlockSpec((H, CB), lambda i: (0, i)),
            pl.BlockSpec((1, CB), lambda i: (0, i)),
        ],
        out_specs=pl.BlockSpec((N, CB), lambda i: (0, i)),
        out_shape=jax.ShapeDtypeStruct((N, 4 * N), jnp.float32),
        scratch_shapes=[pltpu.VMEM((N, H), jnp.float32)],
        compiler_params=_VMEM_PARAMS,
    )(adj, u2, dinv, g2b, f1w, f1b, f2w, f2b, ow, ob)


def kernel(x, edge_index, sage_wl, sage_wr, sage_b, gcn1_w, gcn1_b,
           gcn2_w, gcn2_b, fc1_w, fc1_b, fc2_w, fc2_b, out_w, out_b):
    adj = _adj()(edge_index, jnp.zeros((RH, CW), jnp.float32))
    u1, dinv = _t1_call(adj, x, sage_wl, sage_wr,
                        sage_b.reshape(1, 2 * H), gcn1_w)
    u2 = _t2_call(adj, u1, dinv, gcn1_b.reshape(1, 2 * H), gcn2_w)
    out = _t3_call(adj, u2, dinv, gcn2_b.reshape(1, H),
                   fc1_w, fc1_b.reshape(1, H), fc2_w, fc2_b.reshape(1, H),
                   out_w, out_b.reshape(1, 4 * N))
    return out.reshape(N, 4, N)
